# baseline (speedup 1.0000x reference)
import functools

import jax
import jax.numpy as jnp
from jax import lax
from jax.experimental import pallas as pl
from jax.experimental.pallas import tpu as pltpu
from jax.experimental.pallas import tpu_sc as plsc

_B, _C, _W, _H, _D, _V = 64, 256, 16, 16, 64, 8192
_HW = _W * _H
_EPS = 1e-6
_NC, _NS, _L = 2, 16, 16
_NW = _NC * _NS
_CHUNKS = _V // _L

_NSPLIT = 2
_BPS = _B // _NSPLIT
_TCB = 8


def _tc_gram_body(x_ref, nw_ref, w2t_ref, bfr_ref, btr_ref, g_ref):
    for j in range(_TCB):
        xs = x_ref[j]
        y = xs * nw_ref[...]
        ab = jnp.dot(y, w2t_ref[...], preferred_element_type=jnp.float32)
        ms = jnp.mean(xs * xs)
        scale = lax.rsqrt(ms + _EPS)
        ft = ab[:, :_D] * scale + bfr_ref[...]
        tt = ab[:, _D:] * scale + btr_ref[...]
        g_ref[j] = lax.dot_general(ft, tt, (((1,), (1,)), ((), ())),
                                   preferred_element_type=jnp.float32)


def _tc_gram(xcl, nwcl, w2t, bfr, btr):
    return pl.pallas_call(
        _tc_gram_body,
        grid=(_BPS // _TCB,),
        in_specs=[
            pl.BlockSpec((_TCB, _HW, _C), lambda b: (b, 0, 0)),
            pl.BlockSpec((_HW, _C), lambda b: (0, 0)),
            pl.BlockSpec((_C, 2 * _D), lambda b: (0, 0)),
            pl.BlockSpec((1, _D), lambda b: (0, 0)),
            pl.BlockSpec((1, _D), lambda b: (0, 0)),
        ],
        out_specs=pl.BlockSpec((_TCB, _HW, _HW), lambda b: (b, 0, 0)),
        out_shape=jax.ShapeDtypeStruct((_BPS, _HW, _HW), jnp.float32),
    )(xcl, nwcl, w2t, bfr, btr)


@functools.partial(
    pl.kernel,
    out_type=jax.ShapeDtypeStruct((_BPS, _V), jnp.float32),
    mesh=plsc.VectorSubcoreMesh(core_axis_name="c", subcore_axis_name="s"),
    compiler_params=pltpu.CompilerParams(needs_layout_passes=False),
    scratch_types=[
        pltpu.VMEM((_HW, _HW), jnp.float32),
        pltpu.VMEM((_V,), jnp.float32),
        pltpu.VMEM((_V,), jnp.int32),
        pltpu.VMEM((_V,), jnp.int32),
        pltpu.VMEM((_V,), jnp.int32),
        pltpu.VMEM((_V,), jnp.float32),
        pltpu.VMEM((_L,), jnp.float32),
        pltpu.SemaphoreType.DMA,
    ],
)
def _sc_score(g_hbm, from_hbm, to_hbm, promo_hbm, pb_hbm, out_hbm,
              tab_v, out_v, p_v, q_v, r_v, pbm_v, pb_v, sem):
    wid = lax.axis_index("s") * _NC + lax.axis_index("c")
    cp = pltpu.async_copy(g_hbm.at[wid], tab_v, sem)
    pltpu.sync_copy(from_hbm, p_v)
    pltpu.sync_copy(to_hbm, q_v)
    pltpu.sync_copy(promo_hbm, r_v)
    pltpu.sync_copy(pb_hbm, pb_v)

    @plsc.parallel_loop(0, _CHUNKS, unroll=8)
    def _(i):
        sl = pl.ds(i * _L, _L)
        pbm_v[sl] = plsc.load_gather(pb_v, [r_v[sl]])

    cp.wait()

    @plsc.parallel_loop(0, _CHUNKS, unroll=16)
    def _(i):
        sl = pl.ds(i * _L, _L)
        out_v[sl] = plsc.load_gather(tab_v, [p_v[sl], q_v[sl]]) + pbm_v[sl]

    pltpu.sync_copy(out_v, out_hbm.at[wid])


def kernel(x, from_idx, to_idx, promo_idx, norm_weight, Wf, bf, Wt, bt, promo_bias):
    nwcl = jnp.transpose(norm_weight.reshape(_C, _HW), (1, 0))
    w2t = jnp.concatenate([Wf, Wt], axis=0).T
    bfr = bf.reshape(1, _D)
    btr = bt.reshape(1, _D)
    pb16 = jnp.pad(promo_bias, (0, _L - promo_bias.shape[0]))
    outs = []
    for s in range(_NSPLIT):
        xh = x[s * _BPS:(s + 1) * _BPS]
        xcl = jnp.transpose(xh.reshape(_BPS, _C, _HW), (0, 2, 1))
        gs = _tc_gram(xcl, nwcl, w2t, bfr, btr)
        outs.append(_sc_score(gs, from_idx, to_idx, promo_idx, pb16))
    return jnp.concatenate(outs, axis=0)


# speedup vs baseline: 1.2689x; 1.2689x over previous
import functools

import jax
import jax.numpy as jnp
from jax import lax
from jax.experimental import pallas as pl
from jax.experimental.pallas import tpu as pltpu
from jax.experimental.pallas import tpu_sc as plsc

_B, _C, _W, _H, _D, _V = 64, 256, 16, 16, 64, 8192
_HW = _W * _H
_EPS = 1e-6
_NC, _NS, _L = 2, 16, 16
_NW = _NC * _NS
_CHUNKS = _V // _L

_NSPLIT = 2
_BPS = _B // _NSPLIT
_TCB = 8


def _tc_gram_body(x_ref, nw_ref, w2t_ref, bfr_ref, btr_ref, g_ref):
    for j in range(_TCB):
        xs = x_ref[j]
        y = xs * nw_ref[...]
        ab = jnp.dot(y, w2t_ref[...], preferred_element_type=jnp.float32)
        ms = jnp.mean(xs * xs)
        scale = lax.rsqrt(ms + _EPS)
        ft = ab[:, :_D] * scale + bfr_ref[...]
        tt = ab[:, _D:] * scale + btr_ref[...]
        g_ref[j] = lax.dot_general(ft, tt, (((1,), (1,)), ((), ())),
                                   preferred_element_type=jnp.float32)


def _tc_gram(xcl, nwcl, w2t, bfr, btr, split):
    off = split * (_BPS // _TCB)
    return pl.pallas_call(
        _tc_gram_body,
        grid=(_BPS // _TCB,),
        in_specs=[
            pl.BlockSpec((_TCB, _HW, _C), lambda b: (off + b, 0, 0)),
            pl.BlockSpec((_HW, _C), lambda b: (0, 0)),
            pl.BlockSpec((_C, 2 * _D), lambda b: (0, 0)),
            pl.BlockSpec((1, _D), lambda b: (0, 0)),
            pl.BlockSpec((1, _D), lambda b: (0, 0)),
        ],
        out_specs=pl.BlockSpec((_TCB, _HW, _HW), lambda b: (b, 0, 0)),
        out_shape=jax.ShapeDtypeStruct((_BPS, _HW, _HW), jnp.float32),
    )(xcl, nwcl, w2t, bfr, btr)


@functools.partial(
    pl.kernel,
    out_type=jax.ShapeDtypeStruct((_BPS, _V), jnp.float32),
    mesh=plsc.VectorSubcoreMesh(core_axis_name="c", subcore_axis_name="s"),
    compiler_params=pltpu.CompilerParams(needs_layout_passes=False),
    scratch_types=[
        pltpu.VMEM((_HW, _HW), jnp.float32),
        pltpu.VMEM((_V,), jnp.float32),
        pltpu.VMEM((_V,), jnp.int32),
        pltpu.VMEM((_V,), jnp.int32),
        pltpu.VMEM((_V,), jnp.int32),
        pltpu.VMEM((_V,), jnp.float32),
        pltpu.VMEM((_L,), jnp.float32),
        pltpu.SemaphoreType.DMA,
        pltpu.SemaphoreType.DMA,
    ],
)
def _sc_score(g_hbm, from_hbm, to_hbm, promo_hbm, pb_hbm, out_hbm,
              tab_v, out_v, p_v, q_v, r_v, pbm_v, pb_v, semt, semi):
    wid = lax.axis_index("s") * _NC + lax.axis_index("c")
    cpt = pltpu.async_copy(g_hbm.at[wid], tab_v, semt)
    cp1 = pltpu.async_copy(from_hbm, p_v, semi)
    cp2 = pltpu.async_copy(to_hbm, q_v, semi)
    cp3 = pltpu.async_copy(promo_hbm, r_v, semi)
    cp4 = pltpu.async_copy(pb_hbm, pb_v, semi)
    cp1.wait()
    cp2.wait()
    cp3.wait()
    cp4.wait()

    @plsc.parallel_loop(0, _CHUNKS, unroll=8)
    def _(i):
        sl = pl.ds(i * _L, _L)
        pbm_v[sl] = plsc.load_gather(pb_v, [r_v[sl]])

    cpt.wait()

    @plsc.parallel_loop(0, _CHUNKS, unroll=16)
    def _(i):
        sl = pl.ds(i * _L, _L)
        out_v[sl] = plsc.load_gather(tab_v, [p_v[sl], q_v[sl]]) + pbm_v[sl]

    pltpu.sync_copy(out_v, out_hbm.at[wid])


def kernel(x, from_idx, to_idx, promo_idx, norm_weight, Wf, bf, Wt, bt, promo_bias):
    nwcl = jnp.transpose(norm_weight.reshape(_C, _HW), (1, 0))
    w2t = jnp.concatenate([Wf, Wt], axis=0).T
    bfr = bf.reshape(1, _D)
    btr = bt.reshape(1, _D)
    pb16 = jnp.pad(promo_bias, (0, _L - promo_bias.shape[0]))
    xcl = jnp.transpose(x.reshape(_B, _C, _HW), (0, 2, 1))
    outs = []
    for s in range(_NSPLIT):
        gs = _tc_gram(xcl, nwcl, w2t, bfr, btr, s)
        outs.append(_sc_score(gs, from_idx, to_idx, promo_idx, pb16))
    return jnp.concatenate(outs, axis=0)


# speedup vs baseline: 1.4414x; 1.1359x over previous
import functools

import jax
import jax.numpy as jnp
from jax import lax
from jax.experimental import pallas as pl
from jax.experimental.pallas import tpu as pltpu
from jax.experimental.pallas import tpu_sc as plsc

_B, _C, _W, _H, _D, _V = 64, 256, 16, 16, 64, 8192
_HW = _W * _H
_EPS = 1e-6
_NC, _NS, _L = 2, 16, 16
_NW = _NC * _NS
_CHUNKS = _V // _L

_NSPLIT = 1
_BPS = _B // _NSPLIT
_TCB = 8


def _tc_gram_body(x_ref, nw_ref, w2t_ref, bfr_ref, btr_ref, g_ref):
    for j in range(_TCB):
        xs = x_ref[j]
        y = xs * nw_ref[...]
        ab = jnp.dot(y, w2t_ref[...], preferred_element_type=jnp.float32)
        ms = jnp.mean(xs * xs)
        scale = lax.rsqrt(ms + _EPS)
        ft = ab[:, :_D] * scale + bfr_ref[...]
        tt = ab[:, _D:] * scale + btr_ref[...]
        g_ref[j] = lax.dot_general(ft, tt, (((1,), (1,)), ((), ())),
                                   preferred_element_type=jnp.float32)


def _tc_gram(xcl, nwcl, w2t, bfr, btr, split):
    off = split * (_BPS // _TCB)
    return pl.pallas_call(
        _tc_gram_body,
        grid=(_BPS // _TCB,),
        in_specs=[
            pl.BlockSpec((_TCB, _HW, _C), lambda b: (off + b, 0, 0)),
            pl.BlockSpec((_HW, _C), lambda b: (0, 0)),
            pl.BlockSpec((_C, 2 * _D), lambda b: (0, 0)),
            pl.BlockSpec((1, _D), lambda b: (0, 0)),
            pl.BlockSpec((1, _D), lambda b: (0, 0)),
        ],
        out_specs=pl.BlockSpec((_TCB, _HW, _HW), lambda b: (b, 0, 0)),
        out_shape=jax.ShapeDtypeStruct((_BPS, _HW, _HW), jnp.float32),
    )(xcl, nwcl, w2t, bfr, btr)


@functools.partial(
    pl.kernel,
    out_type=jax.ShapeDtypeStruct((_BPS, _V), jnp.float32),
    mesh=plsc.VectorSubcoreMesh(core_axis_name="c", subcore_axis_name="s"),
    compiler_params=pltpu.CompilerParams(needs_layout_passes=False),
    scratch_types=[
        pltpu.VMEM((_HW, _HW), jnp.float32),
        pltpu.VMEM((_V,), jnp.float32),
        pltpu.VMEM((_V,), jnp.int32),
        pltpu.VMEM((_V,), jnp.int32),
        pltpu.VMEM((_V,), jnp.int32),
        pltpu.VMEM((_V,), jnp.float32),
        pltpu.VMEM((_L,), jnp.float32),
        pltpu.SemaphoreType.DMA,
        pltpu.SemaphoreType.DMA,
    ],
)
def _sc_score(g_hbm, from_hbm, to_hbm, promo_hbm, pb_hbm, out_hbm,
              tab_v, out_v, p_v, q_v, r_v, pbm_v, pb_v, semt, semi):
    wid = lax.axis_index("s") * _NC + lax.axis_index("c")
    bpt = _BPS // _NW
    b0 = wid * bpt
    cpt = pltpu.async_copy(g_hbm.at[b0], tab_v, semt)
    cp1 = pltpu.async_copy(from_hbm, p_v, semi)
    cp2 = pltpu.async_copy(to_hbm, q_v, semi)
    cp3 = pltpu.async_copy(promo_hbm, r_v, semi)
    cp4 = pltpu.async_copy(pb_hbm, pb_v, semi)
    cp1.wait()
    cp2.wait()
    cp3.wait()
    cp4.wait()

    @plsc.parallel_loop(0, _CHUNKS, unroll=8)
    def _(i):
        sl = pl.ds(i * _L, _L)
        pbm_v[sl] = plsc.load_gather(pb_v, [r_v[sl]])

    for j in range(bpt):
        cpt.wait()

        @plsc.parallel_loop(0, _CHUNKS, unroll=16)
        def _(i):
            sl = pl.ds(i * _L, _L)
            out_v[sl] = plsc.load_gather(tab_v, [p_v[sl], q_v[sl]]) + pbm_v[sl]

        if j + 1 < bpt:
            cpt = pltpu.async_copy(g_hbm.at[b0 + j + 1], tab_v, semt)
        pltpu.sync_copy(out_v, out_hbm.at[b0 + j])


def kernel(x, from_idx, to_idx, promo_idx, norm_weight, Wf, bf, Wt, bt, promo_bias):
    nwcl = jnp.transpose(norm_weight.reshape(_C, _HW), (1, 0))
    w2t = jnp.concatenate([Wf, Wt], axis=0).T
    bfr = bf.reshape(1, _D)
    btr = bt.reshape(1, _D)
    pb16 = jnp.pad(promo_bias, (0, _L - promo_bias.shape[0]))
    xcl = jnp.transpose(x.reshape(_B, _C, _HW), (0, 2, 1))
    outs = []
    for s in range(_NSPLIT):
        gs = _tc_gram(xcl, nwcl, w2t, bfr, btr, s)
        outs.append(_sc_score(gs, from_idx, to_idx, promo_idx, pb16))
    return outs[0] if _NSPLIT == 1 else jnp.concatenate(outs, axis=0)


# speedup vs baseline: 1.5001x; 1.0408x over previous
import functools

import jax
import jax.numpy as jnp
from jax import lax
from jax.experimental import pallas as pl
from jax.experimental.pallas import tpu as pltpu
from jax.experimental.pallas import tpu_sc as plsc

_B, _C, _W, _H, _D, _V = 64, 256, 16, 16, 64, 8192
_HW = _W * _H
_EPS = 1e-6
_NC, _NS, _L = 2, 16, 16
_NW = _NC * _NS
_CHUNKS = _V // _L

_NSPLIT = 1
_BPS = _B // _NSPLIT
_TCB = 16


def _tc_gram_body(x_ref, nw_ref, w2t_ref, bfr_ref, btr_ref, g_ref):
    for j in range(_TCB):
        xs = x_ref[j]
        y = xs * nw_ref[...]
        ab = jnp.dot(y, w2t_ref[...], preferred_element_type=jnp.float32)
        ms = jnp.mean(xs * xs)
        scale = lax.rsqrt(ms + _EPS)
        ft = ab[:, :_D] * scale + bfr_ref[...]
        tt = ab[:, _D:] * scale + btr_ref[...]
        g_ref[j] = lax.dot_general(ft, tt, (((1,), (1,)), ((), ())),
                                   preferred_element_type=jnp.float32)


def _tc_gram(xcl, nwcl, w2t, bfr, btr, split):
    off = split * (_BPS // _TCB)
    return pl.pallas_call(
        _tc_gram_body,
        grid=(_BPS // _TCB,),
        in_specs=[
            pl.BlockSpec((_TCB, _HW, _C), lambda b: (off + b, 0, 0)),
            pl.BlockSpec((_HW, _C), lambda b: (0, 0)),
            pl.BlockSpec((_C, 2 * _D), lambda b: (0, 0)),
            pl.BlockSpec((1, _D), lambda b: (0, 0)),
            pl.BlockSpec((1, _D), lambda b: (0, 0)),
        ],
        out_specs=pl.BlockSpec((_TCB, _HW, _HW), lambda b: (b, 0, 0)),
        out_shape=jax.ShapeDtypeStruct((_BPS, _HW, _HW), jnp.float32),
    )(xcl, nwcl, w2t, bfr, btr)


@functools.partial(
    pl.kernel,
    out_type=jax.ShapeDtypeStruct((_BPS, _V), jnp.float32),
    mesh=plsc.VectorSubcoreMesh(core_axis_name="c", subcore_axis_name="s"),
    compiler_params=pltpu.CompilerParams(needs_layout_passes=False),
    scratch_types=[
        pltpu.VMEM((_HW, _HW), jnp.float32),
        pltpu.VMEM((_V,), jnp.float32),
        pltpu.VMEM((_V,), jnp.int32),
        pltpu.VMEM((_V,), jnp.int32),
        pltpu.VMEM((_V,), jnp.int32),
        pltpu.VMEM((_V,), jnp.float32),
        pltpu.VMEM((_L,), jnp.float32),
        pltpu.SemaphoreType.DMA,
        pltpu.SemaphoreType.DMA,
    ],
)
def _sc_score(g_hbm, from_hbm, to_hbm, promo_hbm, pb_hbm, out_hbm,
              tab_v, out_v, p_v, q_v, r_v, pbm_v, pb_v, semt, semi):
    wid = lax.axis_index("s") * _NC + lax.axis_index("c")
    bpt = _BPS // _NW
    b0 = wid * bpt
    cpt = pltpu.async_copy(g_hbm.at[b0], tab_v, semt)
    cp1 = pltpu.async_copy(from_hbm, p_v, semi)
    cp2 = pltpu.async_copy(to_hbm, q_v, semi)
    cp3 = pltpu.async_copy(promo_hbm, r_v, semi)
    cp4 = pltpu.async_copy(pb_hbm, pb_v, semi)
    cp1.wait()
    cp2.wait()
    cp3.wait()
    cp4.wait()

    @plsc.parallel_loop(0, _CHUNKS, unroll=8)
    def _(i):
        sl = pl.ds(i * _L, _L)
        pbm_v[sl] = plsc.load_gather(pb_v, [r_v[sl]])

    for j in range(bpt):
        cpt.wait()

        @plsc.parallel_loop(0, _CHUNKS, unroll=16)
        def _(i):
            sl = pl.ds(i * _L, _L)
            out_v[sl] = plsc.load_gather(tab_v, [p_v[sl], q_v[sl]]) + pbm_v[sl]

        if j + 1 < bpt:
            cpt = pltpu.async_copy(g_hbm.at[b0 + j + 1], tab_v, semt)
        pltpu.sync_copy(out_v, out_hbm.at[b0 + j])


def kernel(x, from_idx, to_idx, promo_idx, norm_weight, Wf, bf, Wt, bt, promo_bias):
    nwcl = jnp.transpose(norm_weight.reshape(_C, _HW), (1, 0))
    w2t = jnp.concatenate([Wf, Wt], axis=0).T
    bfr = bf.reshape(1, _D)
    btr = bt.reshape(1, _D)
    pb16 = jnp.pad(promo_bias, (0, _L - promo_bias.shape[0]))
    xcl = jnp.transpose(x.reshape(_B, _C, _HW), (0, 2, 1))
    outs = []
    for s in range(_NSPLIT):
        gs = _tc_gram(xcl, nwcl, w2t, bfr, btr, s)
        outs.append(_sc_score(gs, from_idx, to_idx, promo_idx, pb16))
    return outs[0] if _NSPLIT == 1 else jnp.concatenate(outs, axis=0)
